# baseline (speedup 1.0000x reference)
import functools

import jax
import jax.numpy as jnp
from jax import lax
from jax.experimental import pallas as pl
from jax.experimental.pallas import tpu as pltpu
from jax.experimental.pallas import tpu_sc as plsc

N_NODE = 100000
HIDDEN = 128
MAX_LEN = 200
BATCH = 1024
SEQ_LEN = 50
TOTAL = BATCH * SEQ_LEN
SCALE = 12.0

NC = 2
NS = 16
NW = NC * NS
TPW = TOTAL // NW
SUB = 64
NSUB = TPW // SUB
BIG = 320
NBIG = TPW // BIG
SPB = BIG // SUB

TB = 2048
NTB = TOTAL // TB
POSP = 208
RB = 32


def _sc_gather_body(emb_hbm, ids2_hbm, seg2_hbm, zeros_hbm,
                    hidden_hbm, sums_hbm, cnts_hbm,
                    idx_v, seg_v, rows_v, ones_v, shsum, shcnt, sem):
    c = lax.axis_index("c")
    s = lax.axis_index("s")
    wid = c * NS + s
    base = wid * TPW

    one16 = jnp.ones((16,), jnp.float32)

    def _fill(i, _):
        ones_v[i // 8, pl.ds((i % 8) * 16, 16)] = one16
        return 0

    lax.fori_loop(0, SUB * 8, _fill, 0)

    seg_lo = s * (BATCH // NS)
    pltpu.sync_copy(zeros_hbm.at[pl.ds(seg_lo, BATCH // NS)],
                    shsum.at[pl.ds(seg_lo, BATCH // NS)])
    pltpu.sync_copy(zeros_hbm.at[pl.ds(seg_lo, BATCH // NS)],
                    shcnt.at[pl.ds(seg_lo, BATCH // NS)])

    pltpu.sync_copy(ids2_hbm.at[wid], idx_v)
    pltpu.sync_copy(seg2_hbm.at[wid], seg_v)

    plsc.subcore_barrier()

    for big in range(NBIG):
        tok0 = base + big * BIG
        cps = []
        for k in range(SPB):
            j = big * SPB + k
            cps.append(pltpu.async_copy(emb_hbm.at[idx_v.at[j]],
                                        rows_v.at[pl.ds(k * SUB, SUB)], sem))
        for cp in cps:
            cp.wait()
        pltpu.sync_copy(rows_v, hidden_hbm.at[pl.ds(tok0, BIG)])
        for k in range(SPB):
            j = big * SPB + k
            pltpu.sync_copy(rows_v.at[pl.ds(k * SUB, SUB)],
                            shsum.at[seg_v.at[j]], add=True)
            pltpu.sync_copy(ones_v, shcnt.at[seg_v.at[j]], add=True)

    plsc.subcore_barrier()

    pltpu.sync_copy(shsum.at[pl.ds(seg_lo, BATCH // NS)],
                    sums_hbm.at[c, pl.ds(seg_lo, BATCH // NS)])
    pltpu.sync_copy(shcnt.at[pl.ds(seg_lo, BATCH // NS)],
                    cnts_hbm.at[c, pl.ds(seg_lo, BATCH // NS)])


def _sc_gather(emb_table, ids2, seg2, zeros, interpret=False):
    return pl.kernel(
        _sc_gather_body,
        out_type=(
            jax.ShapeDtypeStruct((TOTAL, HIDDEN), jnp.float32),
            jax.ShapeDtypeStruct((NC, BATCH, HIDDEN), jnp.float32),
            jax.ShapeDtypeStruct((NC, BATCH, HIDDEN), jnp.float32),
        ),
        mesh=plsc.VectorSubcoreMesh(core_axis_name="c", subcore_axis_name="s",
                                    num_cores=NC, num_subcores=NS),
        scratch_types=[
            pltpu.VMEM((NSUB, SUB), jnp.int32),
            pltpu.VMEM((NSUB, SUB), jnp.int32),
            pltpu.VMEM((BIG, HIDDEN), jnp.float32),
            pltpu.VMEM((SUB, HIDDEN), jnp.float32),
            pltpu.VMEM_SHARED((BATCH, HIDDEN), jnp.float32),
            pltpu.VMEM_SHARED((BATCH, HIDDEN), jnp.float32),
            pltpu.SemaphoreType.DMA,
        ],
        interpret=interpret,
    )(emb_table, ids2, seg2, zeros)


def _attn_body(hid_ref, seg_ref, pos_ref, sums_ref, cnts_ref, pt_ref,
               wpos_ref, bpos_ref, w1_ref, b1_ref, w2_ref, b2_ref,
               wq_ref, bq_ref, out_ref, m1_s, pt2_s):
    i = pl.program_id(0)

    @pl.when(i == 0)
    def _():
        cnt = (cnts_ref[0] + cnts_ref[1])[:, 0:1]
        sums = sums_ref[0] + sums_ref[1]
        means = sums / jnp.maximum(cnt, 1.0)
        m1 = jax.lax.dot_general(
            means, w1_ref[...], (((1,), (0,)), ((), ())),
            preferred_element_type=jnp.float32,
            precision=jax.lax.Precision.HIGHEST) + b1_ref[...]
        m1_s[0] = m1.astype(jnp.bfloat16)
        m1_s[1] = (m1 - m1_s[0].astype(jnp.float32)).astype(jnp.bfloat16)
        pt2 = jax.lax.dot_general(
            pt_ref[...], wpos_ref[pl.ds(HIDDEN, HIDDEN), :],
            (((1,), (0,)), ((), ())), preferred_element_type=jnp.float32,
            precision=jax.lax.Precision.HIGHEST)
        pt2_hi = pt2.astype(jnp.bfloat16)
        pt2_s[0] = pt2_hi
        pt2_s[1] = (pt2 - pt2_hi.astype(jnp.float32)).astype(jnp.bfloat16)

    h = hid_ref[...]
    seg = seg_ref[...].reshape(TB, 1)
    pos = pos_ref[...].reshape(TB, 1)

    oh_seg = (seg == lax.broadcasted_iota(jnp.int32, (TB, BATCH), 1)
              ).astype(jnp.bfloat16)
    oh_pos = (pos == lax.broadcasted_iota(jnp.int32, (TB, POSP), 1)
              ).astype(jnp.bfloat16)

    def _oh_dot(oh, tab_ref):
        hi = jax.lax.dot_general(oh, tab_ref[0], (((1,), (0,)), ((), ())),
                                 preferred_element_type=jnp.float32)
        lo = jax.lax.dot_general(oh, tab_ref[1], (((1,), (0,)), ((), ())),
                                 preferred_element_type=jnp.float32)
        return hi + lo

    mrow = _oh_dot(oh_seg, m1_s)
    posw = _oh_dot(oh_pos, pt2_s)
    hw = jax.lax.dot_general(h, wpos_ref[pl.ds(0, HIDDEN), :],
                             (((1,), (0,)), ((), ())),
                             preferred_element_type=jnp.float32,
            precision=jax.lax.Precision.HIGHEST)
    ph = jnp.tanh(hw + posw + bpos_ref[...])
    phw = jax.lax.dot_general(ph, w2_ref[...], (((1,), (0,)), ((), ())),
                              preferred_element_type=jnp.float32,
            precision=jax.lax.Precision.HIGHEST)
    z = jax.nn.sigmoid(mrow + phw + b2_ref[...])
    alpha = jnp.sum(z * wq_ref[...], axis=1, keepdims=True) + bq_ref[...]
    sg = alpha * h
    sg_hi = sg.astype(jnp.bfloat16)
    sg_lo = (sg - sg_hi.astype(jnp.float32)).astype(jnp.bfloat16)
    contrib = (
        jax.lax.dot_general(oh_seg, sg_hi, (((0,), (0,)), ((), ())),
                            preferred_element_type=jnp.float32)
        + jax.lax.dot_general(oh_seg, sg_lo, (((0,), (0,)), ((), ())),
                              preferred_element_type=jnp.float32))

    @pl.when(i == 0)
    def _():
        out_ref[...] = contrib

    @pl.when(i > 0)
    def _():
        out_ref[...] += contrib


def _attn(hidden, seg_col, pos_col, sums, cnts, pt_pad, W_pos, b_pos,
          W1, b1, W2, b2, wq_row, bq, interpret=False):
    full = lambda shape: pl.BlockSpec(shape, lambda i: tuple(0 for _ in shape))
    return pl.pallas_call(
        _attn_body,
        grid=(NTB,),
        in_specs=[
            pl.BlockSpec((TB, HIDDEN), lambda i: (i, 0)),
            pl.BlockSpec((1, TB, 1), lambda i: (i, 0, 0)),
            pl.BlockSpec((1, TB, 1), lambda i: (i, 0, 0)),
            full((NC, BATCH, HIDDEN)),
            full((NC, BATCH, HIDDEN)),
            full((POSP, HIDDEN)),
            full((2 * HIDDEN, HIDDEN)),
            full((1, HIDDEN)),
            full((HIDDEN, HIDDEN)),
            full((1, HIDDEN)),
            full((HIDDEN, HIDDEN)),
            full((1, HIDDEN)),
            full((1, HIDDEN)),
            full((1, 1)),
        ],
        out_specs=pl.BlockSpec((BATCH, HIDDEN), lambda i: (0, 0)),
        out_shape=jax.ShapeDtypeStruct((BATCH, HIDDEN), jnp.float32),
        scratch_shapes=[
            pltpu.VMEM((2, BATCH, HIDDEN), jnp.bfloat16),
            pltpu.VMEM((2, POSP, HIDDEN), jnp.bfloat16),
        ],
        interpret=interpret,
    )(hidden, seg_col, pos_col, sums, cnts, pt_pad, W_pos, b_pos,
      W1, b1, W2, b2, wq_row, bq)


EC = 4000
NQ = 4
QR = RB // NQ


def _norm_body(emb_ref, out_ref):
    rows = emb_ref[...]
    ss = jnp.sum(rows * rows, axis=1, keepdims=True)
    out_ref[...] = (rows * (SCALE / jnp.maximum(jnp.sqrt(ss), 1e-12))
                    ).astype(jnp.bfloat16)


def _norm(emb_table, interpret=False):
    return pl.pallas_call(
        _norm_body,
        grid=(N_NODE // EC,),
        in_specs=[pl.BlockSpec((EC, HIDDEN), lambda i: (i, 0))],
        out_specs=pl.BlockSpec((EC, HIDDEN), lambda i: (i, 0)),
        out_shape=jax.ShapeDtypeStruct((N_NODE, HIDDEN), jnp.bfloat16),
        interpret=interpret,
    )(emb_table)


def _score_body(fs_ref, embn_hbm, out_hbm, embv, buf, sems, lsem):
    i = pl.program_id(0)
    slot = lax.rem(i, 2)

    @pl.when(i == 0)
    def _():
        pltpu.async_copy(embn_hbm, embv, lsem).wait()

    @pl.when(i >= 2)
    def _():
        for q in range(NQ):
            pltpu.make_async_copy(
                buf.at[slot, pl.ds(q * QR, QR)],
                out_hbm.at[pl.ds((i - 2) * RB + q * QR, QR), :],
                sems.at[slot, q]).wait()

    fs = fs_ref[...]
    ss = jnp.sum(fs * fs, axis=1, keepdims=True)
    fs = (fs / jnp.maximum(jnp.sqrt(ss), 1e-12)).astype(jnp.bfloat16)
    buf[slot] = jax.lax.dot_general(fs, embv[...], (((1,), (1,)), ((), ())),
                                    preferred_element_type=jnp.float32)
    for q in range(NQ):
        pltpu.async_copy(buf.at[slot, pl.ds(q * QR, QR)],
                         out_hbm.at[pl.ds(i * RB + q * QR, QR), :],
                         sems.at[slot, q])

    @pl.when(i == BATCH // RB - 1)
    def _():
        last_slot = (BATCH // RB - 1) % 2
        for sl in (0, 1):
            back = 0 if sl == last_slot else 1
            for q in range(NQ):
                pltpu.make_async_copy(
                    buf.at[sl, pl.ds(q * QR, QR)],
                    out_hbm.at[pl.ds((i - back) * RB + q * QR, QR), :],
                    sems.at[sl, q]).wait()


def _score(final_s, emb_n, interpret=False):
    return pl.pallas_call(
        _score_body,
        grid=(BATCH // RB,),
        in_specs=[
            pl.BlockSpec((RB, HIDDEN), lambda i: (i, 0)),
            pl.BlockSpec(memory_space=pltpu.MemorySpace.HBM),
        ],
        out_specs=pl.BlockSpec(memory_space=pltpu.MemorySpace.HBM),
        out_shape=jax.ShapeDtypeStruct((BATCH, N_NODE), jnp.float32),
        scratch_shapes=[
            pltpu.VMEM((N_NODE, HIDDEN), jnp.bfloat16),
            pltpu.VMEM((2, RB, N_NODE), jnp.float32),
            pltpu.SemaphoreType.DMA((2, NQ)),
            pltpu.SemaphoreType.DMA,
        ],
        interpret=interpret,
    )(final_s, emb_n)


def kernel(item_ids, reverse_pos, segment_ids, emb_table, pos_table,
           W_pos, b_pos, W1, b1, W2, b2, Wq, bq):
    ids2 = item_ids.astype(jnp.int32).reshape(NW, NSUB, SUB)
    seg2 = segment_ids.astype(jnp.int32).reshape(NW, NSUB, SUB)
    zeros = jnp.zeros((BATCH, HIDDEN), jnp.float32)

    hidden, sums, cnts = _sc_gather(emb_table, ids2, seg2, zeros)
    emb_n = _norm(emb_table)

    seg_col = segment_ids.astype(jnp.int32).reshape(NTB, TB, 1)
    pos_col = reverse_pos.astype(jnp.int32).reshape(NTB, TB, 1)
    pt_pad = jnp.pad(pos_table, ((0, POSP - (MAX_LEN + 1)), (0, 0)))
    final_s = _attn(hidden, seg_col, pos_col, sums, cnts, pt_pad, W_pos,
                    b_pos.reshape(1, HIDDEN), W1, b1.reshape(1, HIDDEN),
                    W2, b2.reshape(1, HIDDEN), Wq.reshape(1, HIDDEN),
                    bq.reshape(1, 1))

    return _score(final_s, emb_n)


# speedup vs baseline: 1.0548x; 1.0548x over previous
import functools

import jax
import jax.numpy as jnp
from jax import lax
from jax.experimental import pallas as pl
from jax.experimental.pallas import tpu as pltpu
from jax.experimental.pallas import tpu_sc as plsc

N_NODE = 100000
HIDDEN = 128
MAX_LEN = 200
BATCH = 1024
SEQ_LEN = 50
TOTAL = BATCH * SEQ_LEN
SCALE = 12.0

NC = 2
NS = 16
NW = NC * NS
TPW = TOTAL // NW
SUB = 64
NSUB = TPW // SUB
BIG = 320
NBIG = TPW // BIG
SPB = BIG // SUB

TB = 1024
NTB = TOTAL // TB
POSP = 208
RB = 32


def _sc_gather_body(emb_hbm, ids2_hbm, seg2_hbm, zeros_hbm,
                    hidden_hbm, sums_hbm, cnts_hbm,
                    idx_v, seg_v, rows_v, ones_v, shsum, shcnt, sem):
    c = lax.axis_index("c")
    s = lax.axis_index("s")
    wid = c * NS + s
    base = wid * TPW

    one16 = jnp.ones((16,), jnp.float32)

    def _fill(i, _):
        ones_v[i // 8, pl.ds((i % 8) * 16, 16)] = one16
        return 0

    lax.fori_loop(0, SUB * 8, _fill, 0)

    seg_lo = s * (BATCH // NS)
    pltpu.sync_copy(zeros_hbm.at[pl.ds(seg_lo, BATCH // NS)],
                    shsum.at[pl.ds(seg_lo, BATCH // NS)])
    pltpu.sync_copy(zeros_hbm.at[pl.ds(seg_lo, BATCH // NS)],
                    shcnt.at[pl.ds(seg_lo, BATCH // NS)])

    pltpu.sync_copy(ids2_hbm.at[wid], idx_v)
    pltpu.sync_copy(seg2_hbm.at[wid], seg_v)

    plsc.subcore_barrier()

    for big in range(NBIG):
        tok0 = base + big * BIG
        cps = []
        for k in range(SPB):
            j = big * SPB + k
            cps.append(pltpu.async_copy(emb_hbm.at[idx_v.at[j]],
                                        rows_v.at[pl.ds(k * SUB, SUB)], sem))
        for cp in cps:
            cp.wait()
        pltpu.sync_copy(rows_v, hidden_hbm.at[pl.ds(tok0, BIG)])
        for k in range(SPB):
            j = big * SPB + k
            pltpu.sync_copy(rows_v.at[pl.ds(k * SUB, SUB)],
                            shsum.at[seg_v.at[j]], add=True)
            pltpu.sync_copy(ones_v, shcnt.at[seg_v.at[j]], add=True)

    plsc.subcore_barrier()

    pltpu.sync_copy(shsum.at[pl.ds(seg_lo, BATCH // NS)],
                    sums_hbm.at[c, pl.ds(seg_lo, BATCH // NS)])
    pltpu.sync_copy(shcnt.at[pl.ds(seg_lo, BATCH // NS)],
                    cnts_hbm.at[c, pl.ds(seg_lo, BATCH // NS)])


def _sc_gather(emb_table, ids2, seg2, zeros, interpret=False):
    return pl.kernel(
        _sc_gather_body,
        out_type=(
            jax.ShapeDtypeStruct((TOTAL, HIDDEN), jnp.float32),
            jax.ShapeDtypeStruct((NC, BATCH, HIDDEN), jnp.float32),
            jax.ShapeDtypeStruct((NC, BATCH, HIDDEN), jnp.float32),
        ),
        mesh=plsc.VectorSubcoreMesh(core_axis_name="c", subcore_axis_name="s",
                                    num_cores=NC, num_subcores=NS),
        scratch_types=[
            pltpu.VMEM((NSUB, SUB), jnp.int32),
            pltpu.VMEM((NSUB, SUB), jnp.int32),
            pltpu.VMEM((BIG, HIDDEN), jnp.float32),
            pltpu.VMEM((SUB, HIDDEN), jnp.float32),
            pltpu.VMEM_SHARED((BATCH, HIDDEN), jnp.float32),
            pltpu.VMEM_SHARED((BATCH, HIDDEN), jnp.float32),
            pltpu.SemaphoreType.DMA,
        ],
        interpret=interpret,
    )(emb_table, ids2, seg2, zeros)


def _attn_body(hid_ref, seg_ref, pos_ref, sums_ref, cnts_ref, pt_ref,
               wpos_ref, bpos_ref, w1_ref, b1_ref, w2_ref, b2_ref,
               wq_ref, bq_ref, out_ref, m1_s, pt2_s):
    i = pl.program_id(0)

    @pl.when(i == 0)
    def _():
        cnt = (cnts_ref[0] + cnts_ref[1])[:, 0:1]
        sums = sums_ref[0] + sums_ref[1]
        means = sums / jnp.maximum(cnt, 1.0)
        m1 = jax.lax.dot_general(
            means, w1_ref[...], (((1,), (0,)), ((), ())),
            preferred_element_type=jnp.float32,
            precision=jax.lax.Precision.HIGHEST) + b1_ref[...]
        m1_s[0] = m1.astype(jnp.bfloat16)
        m1_s[1] = (m1 - m1_s[0].astype(jnp.float32)).astype(jnp.bfloat16)
        pt2 = jax.lax.dot_general(
            pt_ref[...], wpos_ref[pl.ds(HIDDEN, HIDDEN), :],
            (((1,), (0,)), ((), ())), preferred_element_type=jnp.float32,
            precision=jax.lax.Precision.HIGHEST)
        pt2_hi = pt2.astype(jnp.bfloat16)
        pt2_s[0] = pt2_hi
        pt2_s[1] = (pt2 - pt2_hi.astype(jnp.float32)).astype(jnp.bfloat16)

    h = hid_ref[...]
    seg = seg_ref[...].reshape(TB, 1)
    pos = pos_ref[...].reshape(TB, 1)

    oh_seg = (seg == lax.broadcasted_iota(jnp.int32, (TB, BATCH), 1)
              ).astype(jnp.bfloat16)
    oh_pos = (pos == lax.broadcasted_iota(jnp.int32, (TB, POSP), 1)
              ).astype(jnp.bfloat16)

    def _oh_dot(oh, tab_ref):
        hi = jax.lax.dot_general(oh, tab_ref[0], (((1,), (0,)), ((), ())),
                                 preferred_element_type=jnp.float32)
        lo = jax.lax.dot_general(oh, tab_ref[1], (((1,), (0,)), ((), ())),
                                 preferred_element_type=jnp.float32)
        return hi + lo

    mrow = _oh_dot(oh_seg, m1_s)
    posw = jax.lax.dot_general(oh_pos, pt2_s[0], (((1,), (0,)), ((), ())),
                               preferred_element_type=jnp.float32)
    hw = jax.lax.dot_general(h, wpos_ref[pl.ds(0, HIDDEN), :],
                             (((1,), (0,)), ((), ())),
                             preferred_element_type=jnp.float32,
            precision=jax.lax.Precision.HIGHEST)
    ph = jnp.tanh(hw + posw + bpos_ref[...])
    phw = jax.lax.dot_general(ph, w2_ref[...], (((1,), (0,)), ((), ())),
                              preferred_element_type=jnp.float32,
            precision=jax.lax.Precision.HIGHEST)
    z = jax.nn.sigmoid(mrow + phw + b2_ref[...])
    alpha = jnp.sum(z * wq_ref[...], axis=1, keepdims=True) + bq_ref[...]
    sg = alpha * h
    sg_hi = sg.astype(jnp.bfloat16)
    sg_lo = (sg - sg_hi.astype(jnp.float32)).astype(jnp.bfloat16)
    contrib = (
        jax.lax.dot_general(oh_seg, sg_hi, (((0,), (0,)), ((), ())),
                            preferred_element_type=jnp.float32)
        + jax.lax.dot_general(oh_seg, sg_lo, (((0,), (0,)), ((), ())),
                              preferred_element_type=jnp.float32))

    @pl.when(i == 0)
    def _():
        out_ref[...] = contrib

    @pl.when(i > 0)
    def _():
        out_ref[...] += contrib


def _attn(hidden, seg_col, pos_col, sums, cnts, pt_pad, W_pos, b_pos,
          W1, b1, W2, b2, wq_row, bq, interpret=False):
    full = lambda shape: pl.BlockSpec(shape, lambda i: tuple(0 for _ in shape))
    return pl.pallas_call(
        _attn_body,
        grid=(NTB,),
        in_specs=[
            pl.BlockSpec((TB, HIDDEN), lambda i: (i, 0)),
            pl.BlockSpec((1, TB, 1), lambda i: (i, 0, 0)),
            pl.BlockSpec((1, TB, 1), lambda i: (i, 0, 0)),
            full((NC, BATCH, HIDDEN)),
            full((NC, BATCH, HIDDEN)),
            full((POSP, HIDDEN)),
            full((2 * HIDDEN, HIDDEN)),
            full((1, HIDDEN)),
            full((HIDDEN, HIDDEN)),
            full((1, HIDDEN)),
            full((HIDDEN, HIDDEN)),
            full((1, HIDDEN)),
            full((1, HIDDEN)),
            full((1, 1)),
        ],
        out_specs=pl.BlockSpec((BATCH, HIDDEN), lambda i: (0, 0)),
        out_shape=jax.ShapeDtypeStruct((BATCH, HIDDEN), jnp.float32),
        scratch_shapes=[
            pltpu.VMEM((2, BATCH, HIDDEN), jnp.bfloat16),
            pltpu.VMEM((2, POSP, HIDDEN), jnp.bfloat16),
        ],
        interpret=interpret,
    )(hidden, seg_col, pos_col, sums, cnts, pt_pad, W_pos, b_pos,
      W1, b1, W2, b2, wq_row, bq)


EC = 4000
NQ = 4
QR = RB // NQ


def _norm_body(emb_ref, out_ref):
    rows = emb_ref[...]
    ss = jnp.sum(rows * rows, axis=1, keepdims=True)
    out_ref[...] = (rows * (SCALE / jnp.maximum(jnp.sqrt(ss), 1e-12))
                    ).astype(jnp.bfloat16)


def _norm(emb_table, interpret=False):
    return pl.pallas_call(
        _norm_body,
        grid=(N_NODE // EC,),
        in_specs=[pl.BlockSpec((EC, HIDDEN), lambda i: (i, 0))],
        out_specs=pl.BlockSpec((EC, HIDDEN), lambda i: (i, 0)),
        out_shape=jax.ShapeDtypeStruct((N_NODE, HIDDEN), jnp.bfloat16),
        interpret=interpret,
    )(emb_table)


def _score_body(fs_ref, embn_hbm, out_hbm, embv, buf, sems, lsem):
    i = pl.program_id(0)
    slot = lax.rem(i, 2)

    @pl.when(i == 0)
    def _():
        pltpu.async_copy(embn_hbm, embv, lsem).wait()

    @pl.when(i >= 2)
    def _():
        for q in range(NQ):
            pltpu.make_async_copy(
                buf.at[slot, pl.ds(q * QR, QR)],
                out_hbm.at[pl.ds((i - 2) * RB + q * QR, QR), :],
                sems.at[slot, q]).wait()

    fs = fs_ref[...]
    ss = jnp.sum(fs * fs, axis=1, keepdims=True)
    fs = (fs / jnp.maximum(jnp.sqrt(ss), 1e-12)).astype(jnp.bfloat16)

    for par in (0, 1):
        @pl.when(slot == par)
        def _():
            buf[par] = jax.lax.dot_general(
                fs, embv[...], (((1,), (1,)), ((), ())),
                preferred_element_type=jnp.float32)
            for q in range(NQ):
                pltpu.async_copy(buf.at[par, pl.ds(q * QR, QR)],
                                 out_hbm.at[pl.ds(i * RB + q * QR, QR), :],
                                 sems.at[par, q])

    @pl.when(i == BATCH // RB - 1)
    def _():
        last_slot = (BATCH // RB - 1) % 2
        for sl in (0, 1):
            back = 0 if sl == last_slot else 1
            for q in range(NQ):
                pltpu.make_async_copy(
                    buf.at[sl, pl.ds(q * QR, QR)],
                    out_hbm.at[pl.ds((i - back) * RB + q * QR, QR), :],
                    sems.at[sl, q]).wait()


def _score(final_s, emb_n, interpret=False):
    return pl.pallas_call(
        _score_body,
        grid=(BATCH // RB,),
        in_specs=[
            pl.BlockSpec((RB, HIDDEN), lambda i: (i, 0)),
            pl.BlockSpec(memory_space=pltpu.MemorySpace.HBM),
        ],
        out_specs=pl.BlockSpec(memory_space=pltpu.MemorySpace.HBM),
        out_shape=jax.ShapeDtypeStruct((BATCH, N_NODE), jnp.float32),
        scratch_shapes=[
            pltpu.VMEM((N_NODE, HIDDEN), jnp.bfloat16),
            pltpu.VMEM((2, RB, N_NODE), jnp.float32),
            pltpu.SemaphoreType.DMA((2, NQ)),
            pltpu.SemaphoreType.DMA,
        ],
        interpret=interpret,
    )(final_s, emb_n)


def kernel(item_ids, reverse_pos, segment_ids, emb_table, pos_table,
           W_pos, b_pos, W1, b1, W2, b2, Wq, bq):
    ids2 = item_ids.astype(jnp.int32).reshape(NW, NSUB, SUB)
    seg2 = segment_ids.astype(jnp.int32).reshape(NW, NSUB, SUB)
    zeros = jnp.zeros((BATCH, HIDDEN), jnp.float32)

    hidden, sums, cnts = _sc_gather(emb_table, ids2, seg2, zeros)
    emb_n = _norm(emb_table)

    seg_col = segment_ids.astype(jnp.int32).reshape(NTB, TB, 1)
    pos_col = reverse_pos.astype(jnp.int32).reshape(NTB, TB, 1)
    pt_pad = jnp.pad(pos_table, ((0, POSP - (MAX_LEN + 1)), (0, 0)))
    final_s = _attn(hidden, seg_col, pos_col, sums, cnts, pt_pad, W_pos,
                    b_pos.reshape(1, HIDDEN), W1, b1.reshape(1, HIDDEN),
                    W2, b2.reshape(1, HIDDEN), Wq.reshape(1, HIDDEN),
                    bq.reshape(1, 1))

    return _score(final_s, emb_n)


# speedup vs baseline: 1.0619x; 1.0067x over previous
import functools

import jax
import jax.numpy as jnp
from jax import lax
from jax.experimental import pallas as pl
from jax.experimental.pallas import tpu as pltpu
from jax.experimental.pallas import tpu_sc as plsc

N_NODE = 100000
HIDDEN = 128
MAX_LEN = 200
BATCH = 1024
SEQ_LEN = 50
TOTAL = BATCH * SEQ_LEN
SCALE = 12.0

NC = 2
NS = 16
NW = NC * NS
TPW = TOTAL // NW
SUB = 64
NSUB = TPW // SUB
BIG = 320
NBIG = TPW // BIG
SPB = BIG // SUB

TB = 1024
NTB = TOTAL // TB
POSP = 208
RB = 32


def _sc_gather_body(emb_hbm, ids2_hbm, seg2_hbm, zeros_hbm,
                    hidden_hbm, sums_hbm, cnts_hbm,
                    idx_v, seg_v, rows_v, ones_v, shsum, shcnt, sem):
    c = lax.axis_index("c")
    s = lax.axis_index("s")
    wid = c * NS + s
    base = wid * TPW

    one16 = jnp.ones((16,), jnp.float32)

    def _fill(i, _):
        ones_v[i // 8, pl.ds((i % 8) * 16, 16)] = one16
        return 0

    lax.fori_loop(0, SUB * 8, _fill, 0)

    seg_lo = s * (BATCH // NS)
    pltpu.sync_copy(zeros_hbm.at[pl.ds(seg_lo, BATCH // NS)],
                    shsum.at[pl.ds(seg_lo, BATCH // NS)])
    pltpu.sync_copy(zeros_hbm.at[pl.ds(seg_lo, BATCH // NS)],
                    shcnt.at[pl.ds(seg_lo, BATCH // NS)])

    pltpu.sync_copy(ids2_hbm.at[wid], idx_v)
    pltpu.sync_copy(seg2_hbm.at[wid], seg_v)

    plsc.subcore_barrier()

    for big in range(NBIG):
        tok0 = base + big * BIG
        cps = []
        for k in range(SPB):
            j = big * SPB + k
            cps.append(pltpu.async_copy(emb_hbm.at[idx_v.at[j]],
                                        rows_v.at[pl.ds(k * SUB, SUB)], sem))
        for cp in cps:
            cp.wait()
        pltpu.sync_copy(rows_v, hidden_hbm.at[pl.ds(tok0, BIG)])
        for k in range(SPB):
            j = big * SPB + k
            pltpu.sync_copy(rows_v.at[pl.ds(k * SUB, SUB)],
                            shsum.at[seg_v.at[j]], add=True)
            pltpu.sync_copy(ones_v, shcnt.at[seg_v.at[j]], add=True)

    plsc.subcore_barrier()

    pltpu.sync_copy(shsum.at[pl.ds(seg_lo, BATCH // NS)],
                    sums_hbm.at[c, pl.ds(seg_lo, BATCH // NS)])
    pltpu.sync_copy(shcnt.at[pl.ds(seg_lo, BATCH // NS)],
                    cnts_hbm.at[c, pl.ds(seg_lo, BATCH // NS)])


def _sc_gather(emb_table, ids2, seg2, zeros, interpret=False):
    return pl.kernel(
        _sc_gather_body,
        out_type=(
            jax.ShapeDtypeStruct((TOTAL, HIDDEN), jnp.float32),
            jax.ShapeDtypeStruct((NC, BATCH, HIDDEN), jnp.float32),
            jax.ShapeDtypeStruct((NC, BATCH, HIDDEN), jnp.float32),
        ),
        mesh=plsc.VectorSubcoreMesh(core_axis_name="c", subcore_axis_name="s",
                                    num_cores=NC, num_subcores=NS),
        scratch_types=[
            pltpu.VMEM((NSUB, SUB), jnp.int32),
            pltpu.VMEM((NSUB, SUB), jnp.int32),
            pltpu.VMEM((BIG, HIDDEN), jnp.float32),
            pltpu.VMEM((SUB, HIDDEN), jnp.float32),
            pltpu.VMEM_SHARED((BATCH, HIDDEN), jnp.float32),
            pltpu.VMEM_SHARED((BATCH, HIDDEN), jnp.float32),
            pltpu.SemaphoreType.DMA,
        ],
        interpret=interpret,
    )(emb_table, ids2, seg2, zeros)


def _attn_body(hid_ref, seg_ref, pos_ref, sums_ref, cnts_ref, pt_ref,
               wpos_ref, bpos_ref, w1_ref, b1_ref, w2_ref, b2_ref,
               wq_ref, bq_ref, out_ref, m1_s, pt2_s, wp1_s, w2_s):
    i = pl.program_id(0)

    @pl.when(i == 0)
    def _():
        cnt = (cnts_ref[0] + cnts_ref[1])[:, 0:1]
        sums = sums_ref[0] + sums_ref[1]
        means = sums / jnp.maximum(cnt, 1.0)
        m1 = jax.lax.dot_general(
            means, w1_ref[...], (((1,), (0,)), ((), ())),
            preferred_element_type=jnp.float32,
            precision=jax.lax.Precision.HIGHEST) + b1_ref[...]
        m1_s[0] = m1.astype(jnp.bfloat16)
        m1_s[1] = (m1 - m1_s[0].astype(jnp.float32)).astype(jnp.bfloat16)
        pt2 = jax.lax.dot_general(
            pt_ref[...], wpos_ref[pl.ds(HIDDEN, HIDDEN), :],
            (((1,), (0,)), ((), ())), preferred_element_type=jnp.float32,
            precision=jax.lax.Precision.HIGHEST)
        pt2_hi = pt2.astype(jnp.bfloat16)
        pt2_s[0] = pt2_hi
        pt2_s[1] = (pt2 - pt2_hi.astype(jnp.float32)).astype(jnp.bfloat16)
        wp1 = wpos_ref[pl.ds(0, HIDDEN), :]
        wp1_s[0] = wp1.astype(jnp.bfloat16)
        wp1_s[1] = (wp1 - wp1_s[0].astype(jnp.float32)).astype(jnp.bfloat16)
        w2 = w2_ref[...]
        w2_s[0] = w2.astype(jnp.bfloat16)
        w2_s[1] = (w2 - w2_s[0].astype(jnp.float32)).astype(jnp.bfloat16)

    h = hid_ref[...]
    seg = seg_ref[...].reshape(TB, 1)
    pos = pos_ref[...].reshape(TB, 1)

    oh_seg = (seg == lax.broadcasted_iota(jnp.int32, (TB, BATCH), 1)
              ).astype(jnp.bfloat16)
    oh_pos = (pos == lax.broadcasted_iota(jnp.int32, (TB, POSP), 1)
              ).astype(jnp.bfloat16)

    def _oh_dot(oh, tab_ref):
        hi = jax.lax.dot_general(oh, tab_ref[0], (((1,), (0,)), ((), ())),
                                 preferred_element_type=jnp.float32)
        lo = jax.lax.dot_general(oh, tab_ref[1], (((1,), (0,)), ((), ())),
                                 preferred_element_type=jnp.float32)
        return hi + lo

    mrow = _oh_dot(oh_seg, m1_s)
    posw = jax.lax.dot_general(oh_pos, pt2_s[0], (((1,), (0,)), ((), ())),
                               preferred_element_type=jnp.float32)
    def _split_dot(x, w_s):
        x_hi = x.astype(jnp.bfloat16)
        x_lo = (x - x_hi.astype(jnp.float32)).astype(jnp.bfloat16)
        return (jax.lax.dot_general(x_hi, w_s[0], (((1,), (0,)), ((), ())),
                                    preferred_element_type=jnp.float32)
                + jax.lax.dot_general(x_hi, w_s[1], (((1,), (0,)), ((), ())),
                                      preferred_element_type=jnp.float32)
                + jax.lax.dot_general(x_lo, w_s[0], (((1,), (0,)), ((), ())),
                                      preferred_element_type=jnp.float32))

    hw = _split_dot(h, wp1_s)
    ph = jnp.tanh(hw + posw + bpos_ref[...])
    phw = _split_dot(ph, w2_s)
    z = jax.nn.sigmoid(mrow + phw + b2_ref[...])
    alpha = jnp.sum(z * wq_ref[...], axis=1, keepdims=True) + bq_ref[...]
    sg = alpha * h
    sg_hi = sg.astype(jnp.bfloat16)
    sg_lo = (sg - sg_hi.astype(jnp.float32)).astype(jnp.bfloat16)
    contrib = (
        jax.lax.dot_general(oh_seg, sg_hi, (((0,), (0,)), ((), ())),
                            preferred_element_type=jnp.float32)
        + jax.lax.dot_general(oh_seg, sg_lo, (((0,), (0,)), ((), ())),
                              preferred_element_type=jnp.float32))

    @pl.when(i == 0)
    def _():
        out_ref[...] = contrib

    @pl.when(i > 0)
    def _():
        out_ref[...] += contrib


def _attn(hidden, seg_col, pos_col, sums, cnts, pt_pad, W_pos, b_pos,
          W1, b1, W2, b2, wq_row, bq, interpret=False):
    full = lambda shape: pl.BlockSpec(shape, lambda i: tuple(0 for _ in shape))
    return pl.pallas_call(
        _attn_body,
        grid=(NTB,),
        in_specs=[
            pl.BlockSpec((TB, HIDDEN), lambda i: (i, 0)),
            pl.BlockSpec((1, TB, 1), lambda i: (i, 0, 0)),
            pl.BlockSpec((1, TB, 1), lambda i: (i, 0, 0)),
            full((NC, BATCH, HIDDEN)),
            full((NC, BATCH, HIDDEN)),
            full((POSP, HIDDEN)),
            full((2 * HIDDEN, HIDDEN)),
            full((1, HIDDEN)),
            full((HIDDEN, HIDDEN)),
            full((1, HIDDEN)),
            full((HIDDEN, HIDDEN)),
            full((1, HIDDEN)),
            full((1, HIDDEN)),
            full((1, 1)),
        ],
        out_specs=pl.BlockSpec((BATCH, HIDDEN), lambda i: (0, 0)),
        out_shape=jax.ShapeDtypeStruct((BATCH, HIDDEN), jnp.float32),
        scratch_shapes=[
            pltpu.VMEM((2, BATCH, HIDDEN), jnp.bfloat16),
            pltpu.VMEM((2, POSP, HIDDEN), jnp.bfloat16),
            pltpu.VMEM((2, HIDDEN, HIDDEN), jnp.bfloat16),
            pltpu.VMEM((2, HIDDEN, HIDDEN), jnp.bfloat16),
        ],
        interpret=interpret,
    )(hidden, seg_col, pos_col, sums, cnts, pt_pad, W_pos, b_pos,
      W1, b1, W2, b2, wq_row, bq)


EC = 4000
NQ = 1
QR = RB // NQ


def _norm_body(emb_ref, out_ref):
    rows = emb_ref[...]
    ss = jnp.sum(rows * rows, axis=1, keepdims=True)
    out_ref[...] = (rows * (SCALE / jnp.maximum(jnp.sqrt(ss), 1e-12))
                    ).astype(jnp.bfloat16)


def _norm(emb_table, interpret=False):
    return pl.pallas_call(
        _norm_body,
        grid=(N_NODE // EC,),
        in_specs=[pl.BlockSpec((EC, HIDDEN), lambda i: (i, 0))],
        out_specs=pl.BlockSpec((EC, HIDDEN), lambda i: (i, 0)),
        out_shape=jax.ShapeDtypeStruct((N_NODE, HIDDEN), jnp.bfloat16),
        interpret=interpret,
    )(emb_table)


def _score_body(fs_ref, embn_hbm, out_hbm, embv, buf, sems, lsem):
    i = pl.program_id(0)
    slot = lax.rem(i, 2)

    @pl.when(i == 0)
    def _():
        pltpu.async_copy(embn_hbm, embv, lsem).wait()

    @pl.when(i >= 2)
    def _():
        for q in range(NQ):
            pltpu.make_async_copy(
                buf.at[slot, pl.ds(q * QR, QR)],
                out_hbm.at[pl.ds((i - 2) * RB + q * QR, QR), :],
                sems.at[slot, q]).wait()

    fs = fs_ref[...]
    ss = jnp.sum(fs * fs, axis=1, keepdims=True)
    fs = (fs / jnp.maximum(jnp.sqrt(ss), 1e-12)).astype(jnp.bfloat16)

    for par in (0, 1):
        @pl.when(slot == par)
        def _():
            buf[par] = jax.lax.dot_general(
                fs, embv[...], (((1,), (1,)), ((), ())),
                preferred_element_type=jnp.float32)
            for q in range(NQ):
                pltpu.async_copy(buf.at[par, pl.ds(q * QR, QR)],
                                 out_hbm.at[pl.ds(i * RB + q * QR, QR), :],
                                 sems.at[par, q])

    @pl.when(i == BATCH // RB - 1)
    def _():
        last_slot = (BATCH // RB - 1) % 2
        for sl in (0, 1):
            back = 0 if sl == last_slot else 1
            for q in range(NQ):
                pltpu.make_async_copy(
                    buf.at[sl, pl.ds(q * QR, QR)],
                    out_hbm.at[pl.ds((i - back) * RB + q * QR, QR), :],
                    sems.at[sl, q]).wait()


def _score(final_s, emb_n, interpret=False):
    return pl.pallas_call(
        _score_body,
        grid=(BATCH // RB,),
        in_specs=[
            pl.BlockSpec((RB, HIDDEN), lambda i: (i, 0)),
            pl.BlockSpec(memory_space=pltpu.MemorySpace.HBM),
        ],
        out_specs=pl.BlockSpec(memory_space=pltpu.MemorySpace.HBM),
        out_shape=jax.ShapeDtypeStruct((BATCH, N_NODE), jnp.float32),
        scratch_shapes=[
            pltpu.VMEM((N_NODE, HIDDEN), jnp.bfloat16),
            pltpu.VMEM((2, RB, N_NODE), jnp.float32),
            pltpu.SemaphoreType.DMA((2, NQ)),
            pltpu.SemaphoreType.DMA,
        ],
        interpret=interpret,
    )(final_s, emb_n)


def kernel(item_ids, reverse_pos, segment_ids, emb_table, pos_table,
           W_pos, b_pos, W1, b1, W2, b2, Wq, bq):
    ids2 = item_ids.astype(jnp.int32).reshape(NW, NSUB, SUB)
    seg2 = segment_ids.astype(jnp.int32).reshape(NW, NSUB, SUB)
    zeros = jnp.zeros((BATCH, HIDDEN), jnp.float32)

    hidden, sums, cnts = _sc_gather(emb_table, ids2, seg2, zeros)
    emb_n = _norm(emb_table)

    seg_col = segment_ids.astype(jnp.int32).reshape(NTB, TB, 1)
    pos_col = reverse_pos.astype(jnp.int32).reshape(NTB, TB, 1)
    pt_pad = jnp.pad(pos_table, ((0, POSP - (MAX_LEN + 1)), (0, 0)))
    final_s = _attn(hidden, seg_col, pos_col, sums, cnts, pt_pad, W_pos,
                    b_pos.reshape(1, HIDDEN), W1, b1.reshape(1, HIDDEN),
                    W2, b2.reshape(1, HIDDEN), Wq.reshape(1, HIDDEN),
                    bq.reshape(1, 1))

    return _score(final_s, emb_n)
